# baseline (speedup 1.0000x reference)
import jax
import jax.numpy as jnp
from jax.experimental import pallas as pl
from jax.experimental.pallas import tpu as pltpu

_TC = 8


def _gates(gx, gh, b_hh, h_prev, H):
    r = jax.nn.sigmoid(gx[:, :H] + gh[:, :H] + b_hh[:, :H])
    z = jax.nn.sigmoid(gx[:, H:2 * H] + gh[:, H:2 * H] + b_hh[:, H:2 * H])
    n = jnp.tanh(gx[:, 2 * H:] + r * (gh[:, 2 * H:] + b_hh[:, 2 * H:]))
    return (1.0 - z) * n + z * h_prev


def _project(embs_ref, oh_ref, rel_emb_ref, wiT_ref, wrxT_ref, bi_ref,
             brx_ref, gi_dst, grel_dst):
    B, Tc, E = embs_ref.shape
    G = gi_dst.shape[2]
    X = embs_ref[...].reshape(B * Tc, E).astype(jnp.bfloat16)
    Gi = jnp.dot(X, wiT_ref[...],
                 preferred_element_type=jnp.float32) + bi_ref[...]
    gi_dst[...] = Gi.reshape(B, Tc, G)
    OH = oh_ref[...].reshape(B * Tc, oh_ref.shape[2])
    relx = jnp.dot(OH, rel_emb_ref[...], preferred_element_type=jnp.float32)
    Grel = jnp.dot(relx.astype(jnp.bfloat16), wrxT_ref[...],
                   preferred_element_type=jnp.float32) + brx_ref[...]
    grel_dst[...] = Grel.reshape(B, Tc, G)


def _chain(parents_sm, red_ref, out_ref, whT_ref, wrhT_ref, bh_ref, brh_ref,
           gi_src, grel_src, t_hi, out_base, Tc, B, H):
    for j in range(Tc):
        t = t_hi - j
        l = Tc - 1 - j
        red_t = red_ref[pl.ds(t, 1), :, :][0]

        gh = jnp.dot(red_t.astype(jnp.bfloat16), whT_ref[...],
                     preferred_element_type=jnp.float32)
        gx = gi_src[:, l, :]
        h = _gates(gx, gh, bh_ref[...], red_t, H)
        out_ref[pl.ds(out_base + l, 1)] = h[None]

        ghr = jnp.dot(h.astype(jnp.bfloat16), wrhT_ref[...],
                      preferred_element_type=jnp.float32)
        gxr = grel_src[:, l, :]
        msg = _gates(gxr, ghr, brh_ref[...], h, H)

        @pl.when(t > 0)
        def _scatter(msg=msg, t=t):
            GRP = 8
            for g in range(0, B, GRP):
                ps = [parents_sm[t, b] for b in range(g, g + GRP)]
                loaded = [red_ref[pl.ds(ps[k], 1), pl.ds(g + k, 1), :]
                          for k in range(GRP)]
                for k in range(GRP):
                    b = g + k
                    red_ref[pl.ds(ps[k], 1), pl.ds(b, 1), :] = (
                        loaded[k] + msg[b:b + 1, :][None])


def _tree_gru_kernel(parents_sm, embs_p_ref, oh_p_ref, embs_1_ref, oh_1_ref,
                     embs_2_ref, oh_2_ref, rel_emb_ref, wiT_ref, whT_ref,
                     wrxT_ref, wrhT_ref, bi_ref, bh_ref, brx_ref, brh_ref,
                     out_ref, red_ref, gia_scr, grela_scr, gib_scr,
                     grelb_scr):
    k = pl.program_id(0)
    n_bodies = pl.num_programs(0)
    B, Tc, E = embs_1_ref.shape
    H = red_ref.shape[2]
    nb = 2 * n_bodies

    @pl.when(k == 0)
    def _init():
        red_ref[...] = jnp.zeros_like(red_ref)
        _project(embs_p_ref, oh_p_ref, rel_emb_ref, wiT_ref, wrxT_ref,
                 bi_ref, brx_ref, gia_scr, grela_scr)

    _project(embs_1_ref, oh_1_ref, rel_emb_ref, wiT_ref, wrxT_ref, bi_ref,
             brx_ref, gib_scr, grelb_scr)
    t_hi1 = (nb - 1 - 2 * k) * Tc + Tc - 1
    _chain(parents_sm, red_ref, out_ref, whT_ref, wrhT_ref, bh_ref, brh_ref,
           gia_scr, grela_scr, t_hi1, Tc, Tc, B, H)

    _project(embs_2_ref, oh_2_ref, rel_emb_ref, wiT_ref, wrxT_ref, bi_ref,
             brx_ref, gia_scr, grela_scr)
    t_hi2 = t_hi1 - Tc
    _chain(parents_sm, red_ref, out_ref, whT_ref, wrhT_ref, bh_ref, brh_ref,
           gib_scr, grelb_scr, t_hi2, 0, Tc, B, H)


def kernel(embs, parents, rels, rel_emb, nw_ih, nw_hh, nb_ih, nb_hh, rw_ih,
           rw_hh, rb_ih, rb_hh):
    B, T, E = embs.shape
    H = nw_hh.shape[1]
    G = 3 * H
    R = rel_emb.shape[0]
    Tc = _TC

    embs_f = embs.astype(jnp.float32)
    onehot = (rels.astype(jnp.int32)[:, :, None]
              == jnp.arange(R, dtype=jnp.int32)
              ).astype(jnp.bfloat16)
    parents_t = parents.astype(jnp.int32).T
    rel_emb_b = rel_emb.astype(jnp.bfloat16)
    wiT = nw_ih.T.astype(jnp.bfloat16)
    whT = nw_hh.T.astype(jnp.bfloat16)
    wrxT = rw_ih.T.astype(jnp.bfloat16)
    wrhT = rw_hh.T.astype(jnp.bfloat16)
    bi = nb_ih.reshape(1, G)
    bh = nb_hh.reshape(1, G)
    brx = rb_ih.reshape(1, G)
    brh = rb_hh.reshape(1, G)

    nb = T // Tc
    n_bodies = nb // 2

    def map_p(k, pref):
        return (0, jnp.where(k == 0, nb - 1, 0), 0)

    def map_1(k, pref):
        return (0, jnp.maximum(nb - 2 - 2 * k, 0), 0)

    def map_2(k, pref):
        return (0, jnp.maximum(nb - 3 - 2 * k, 0), 0)

    def map_out(k, pref):
        return (n_bodies - 1 - k, 0, 0)

    grid_spec = pltpu.PrefetchScalarGridSpec(
        num_scalar_prefetch=1,
        grid=(n_bodies,),
        in_specs=[
            pl.BlockSpec((B, Tc, E), map_p),
            pl.BlockSpec((B, Tc, R), map_p),
            pl.BlockSpec((B, Tc, E), map_1),
            pl.BlockSpec((B, Tc, R), map_1),
            pl.BlockSpec((B, Tc, E), map_2),
            pl.BlockSpec((B, Tc, R), map_2),
            pl.BlockSpec((R, E), lambda k, pref: (0, 0)),
            pl.BlockSpec((E, G), lambda k, pref: (0, 0)),
            pl.BlockSpec((H, G), lambda k, pref: (0, 0)),
            pl.BlockSpec((E, G), lambda k, pref: (0, 0)),
            pl.BlockSpec((H, G), lambda k, pref: (0, 0)),
            pl.BlockSpec((1, G), lambda k, pref: (0, 0)),
            pl.BlockSpec((1, G), lambda k, pref: (0, 0)),
            pl.BlockSpec((1, G), lambda k, pref: (0, 0)),
            pl.BlockSpec((1, G), lambda k, pref: (0, 0)),
        ],
        out_specs=pl.BlockSpec((2 * Tc, B, H), map_out),
        scratch_shapes=[
            pltpu.VMEM((T, B, H), jnp.float32),
            pltpu.VMEM((B, Tc, G), jnp.float32),
            pltpu.VMEM((B, Tc, G), jnp.float32),
            pltpu.VMEM((B, Tc, G), jnp.float32),
            pltpu.VMEM((B, Tc, G), jnp.float32),
        ],
    )
    hs = pl.pallas_call(
        _tree_gru_kernel,
        grid_spec=grid_spec,
        out_shape=jax.ShapeDtypeStruct((T, B, H), jnp.float32),
        compiler_params=pltpu.CompilerParams(
            dimension_semantics=("arbitrary",),
        ),
    )(parents_t, embs_f, onehot, embs_f, onehot, embs_f, onehot,
      rel_emb_b, wiT, whT, wrxT, wrhT, bi, bh, brx, brh)

    return hs.transpose(1, 0, 2)


# speedup vs baseline: 1.4870x; 1.4870x over previous
import jax
import jax.numpy as jnp
from jax.experimental import pallas as pl
from jax.experimental.pallas import tpu as pltpu

_TC = 8


def _gates(gx, gh, b_hh, h_prev, H):
    r = jax.nn.sigmoid(gx[:, :H] + gh[:, :H] + b_hh[:, :H])
    z = jax.nn.sigmoid(gx[:, H:2 * H] + gh[:, H:2 * H] + b_hh[:, H:2 * H])
    n = jnp.tanh(gx[:, 2 * H:] + r * (gh[:, 2 * H:] + b_hh[:, 2 * H:]))
    return (1.0 - z) * n + z * h_prev


def _project(embs_ref, oh_ref, rel_emb_ref, wiT_ref, wrxT_ref, bi_ref,
             brx_ref, gi_dst, grel_dst):
    Tc, B, E = embs_ref.shape
    X = embs_ref[...].reshape(Tc * B, E).astype(jnp.bfloat16)
    gi_dst[...] = jnp.dot(X, wiT_ref[...],
                          preferred_element_type=jnp.float32) + bi_ref[...]
    OH = oh_ref[...].reshape(Tc * B, oh_ref.shape[2])
    relx = jnp.dot(OH, rel_emb_ref[...], preferred_element_type=jnp.float32)
    grel_dst[...] = jnp.dot(relx.astype(jnp.bfloat16), wrxT_ref[...],
                            preferred_element_type=jnp.float32) + brx_ref[...]


def _chain(parents_sm, red_ref, out_ref, whT_ref, wrhT_ref, bh_ref, brh_ref,
           gi_src, grel_src, t_hi, out_base, Tc, B, H):
    for j in range(Tc):
        t = t_hi - j
        l = Tc - 1 - j
        red_t = red_ref[pl.ds(t, 1), :, :][0]

        gh = jnp.dot(red_t.astype(jnp.bfloat16), whT_ref[...],
                     preferred_element_type=jnp.float32)
        gx = gi_src[pl.ds(l * B, B), :]
        h = _gates(gx, gh, bh_ref[...], red_t, H)
        out_ref[pl.ds(out_base + l, 1)] = h[None]

        ghr = jnp.dot(h.astype(jnp.bfloat16), wrhT_ref[...],
                      preferred_element_type=jnp.float32)
        gxr = grel_src[pl.ds(l * B, B), :]
        msg = _gates(gxr, ghr, brh_ref[...], h, H)

        @pl.when(t > 0)
        def _scatter(msg=msg, t=t):
            GRP = 16
            for g in range(0, B, GRP):
                ps = [parents_sm[t, b] for b in range(g, g + GRP)]
                loaded = [red_ref[pl.ds(ps[k], 1), pl.ds(g + k, 1), :]
                          for k in range(GRP)]
                for k in range(GRP):
                    b = g + k
                    red_ref[pl.ds(ps[k], 1), pl.ds(b, 1), :] = (
                        loaded[k] + msg[b:b + 1, :][None])


def _tree_gru_kernel(parents_sm, embs_p_ref, oh_p_ref, embs_1_ref, oh_1_ref,
                     embs_2_ref, oh_2_ref, rel_emb_ref, wiT_ref, whT_ref,
                     wrxT_ref, wrhT_ref, bi_ref, bh_ref, brx_ref, brh_ref,
                     out_ref, red_ref, gia_scr, grela_scr, gib_scr,
                     grelb_scr):
    k = pl.program_id(0)
    n_bodies = pl.num_programs(0)
    Tc, B, E = embs_1_ref.shape
    H = red_ref.shape[2]
    nb = 2 * n_bodies

    @pl.when(k == 0)
    def _init():
        red_ref[...] = jnp.zeros_like(red_ref)
        _project(embs_p_ref, oh_p_ref, rel_emb_ref, wiT_ref, wrxT_ref,
                 bi_ref, brx_ref, gia_scr, grela_scr)

    _project(embs_1_ref, oh_1_ref, rel_emb_ref, wiT_ref, wrxT_ref, bi_ref,
             brx_ref, gib_scr, grelb_scr)
    t_hi1 = (nb - 1 - 2 * k) * Tc + Tc - 1
    _chain(parents_sm, red_ref, out_ref, whT_ref, wrhT_ref, bh_ref, brh_ref,
           gia_scr, grela_scr, t_hi1, Tc, Tc, B, H)

    _project(embs_2_ref, oh_2_ref, rel_emb_ref, wiT_ref, wrxT_ref, bi_ref,
             brx_ref, gia_scr, grela_scr)
    t_hi2 = t_hi1 - Tc
    _chain(parents_sm, red_ref, out_ref, whT_ref, wrhT_ref, bh_ref, brh_ref,
           gib_scr, grelb_scr, t_hi2, 0, Tc, B, H)


def kernel(embs, parents, rels, rel_emb, nw_ih, nw_hh, nb_ih, nb_hh, rw_ih,
           rw_hh, rb_ih, rb_hh):
    B, T, E = embs.shape
    H = nw_hh.shape[1]
    G = 3 * H
    R = rel_emb.shape[0]
    Tc = _TC

    embs_t = embs.astype(jnp.float32).transpose(1, 0, 2)
    rels_t = rels.astype(jnp.int32).T
    onehot_t = (rels_t[:, :, None] == jnp.arange(R, dtype=jnp.int32)
                ).astype(jnp.bfloat16)
    parents_t = parents.astype(jnp.int32).T
    rel_emb_b = rel_emb.astype(jnp.bfloat16)
    wiT = nw_ih.T.astype(jnp.bfloat16)
    whT = nw_hh.T.astype(jnp.bfloat16)
    wrxT = rw_ih.T.astype(jnp.bfloat16)
    wrhT = rw_hh.T.astype(jnp.bfloat16)
    bi = nb_ih.reshape(1, G)
    bh = nb_hh.reshape(1, G)
    brx = rb_ih.reshape(1, G)
    brh = rb_hh.reshape(1, G)

    nb = T // Tc
    n_bodies = nb // 2

    def map_p(k, pref):
        return (jnp.where(k == 0, nb - 1, 0), 0, 0)

    def map_1(k, pref):
        return (jnp.maximum(nb - 2 - 2 * k, 0), 0, 0)

    def map_2(k, pref):
        return (jnp.maximum(nb - 3 - 2 * k, 0), 0, 0)

    def map_out(k, pref):
        return (n_bodies - 1 - k, 0, 0)

    grid_spec = pltpu.PrefetchScalarGridSpec(
        num_scalar_prefetch=1,
        grid=(n_bodies,),
        in_specs=[
            pl.BlockSpec((Tc, B, E), map_p),
            pl.BlockSpec((Tc, B, R), map_p),
            pl.BlockSpec((Tc, B, E), map_1),
            pl.BlockSpec((Tc, B, R), map_1),
            pl.BlockSpec((Tc, B, E), map_2),
            pl.BlockSpec((Tc, B, R), map_2),
            pl.BlockSpec((R, E), lambda k, pref: (0, 0)),
            pl.BlockSpec((E, G), lambda k, pref: (0, 0)),
            pl.BlockSpec((H, G), lambda k, pref: (0, 0)),
            pl.BlockSpec((E, G), lambda k, pref: (0, 0)),
            pl.BlockSpec((H, G), lambda k, pref: (0, 0)),
            pl.BlockSpec((1, G), lambda k, pref: (0, 0)),
            pl.BlockSpec((1, G), lambda k, pref: (0, 0)),
            pl.BlockSpec((1, G), lambda k, pref: (0, 0)),
            pl.BlockSpec((1, G), lambda k, pref: (0, 0)),
        ],
        out_specs=pl.BlockSpec((2 * Tc, B, H), map_out),
        scratch_shapes=[
            pltpu.VMEM((T, B, H), jnp.float32),
            pltpu.VMEM((Tc * B, G), jnp.float32),
            pltpu.VMEM((Tc * B, G), jnp.float32),
            pltpu.VMEM((Tc * B, G), jnp.float32),
            pltpu.VMEM((Tc * B, G), jnp.float32),
        ],
    )
    hs = pl.pallas_call(
        _tree_gru_kernel,
        grid_spec=grid_spec,
        out_shape=jax.ShapeDtypeStruct((T, B, H), jnp.float32),
        compiler_params=pltpu.CompilerParams(
            dimension_semantics=("arbitrary",),
        ),
    )(parents_t, embs_t, onehot_t, embs_t, onehot_t, embs_t, onehot_t,
      rel_emb_b, wiT, whT, wrxT, wrhT, bi, bh, brx, brh)

    return hs.transpose(1, 0, 2)
